# baseline (speedup 1.0000x reference)
import functools

import jax
import jax.numpy as jnp
from jax import lax
from jax.experimental import pallas as pl
from jax.experimental.pallas import tpu as pltpu
from jax.experimental.pallas import tpu_sc as plsc

_B, _L, _D, _O, _E, _T = 4, 2048, 1024, 1024, 8, 256
_N = _L // _T
_S = _B * _N
_SB = 4
_TP = 128
_KP = _S // _SB
_OO = 256
_OH = _O // 2
_NM = _N - 1


def _prep_body(x_ref, wg_ref, w_ref, rw_ref, b_ref, rb_ref,
               logits_ref, wt_ref, dbm_ref):
    k = pl.program_id(0)
    w = w_ref[...]
    rw = rw_ref[...]
    dw = w - rw[None]
    sdw = jnp.sum(dw, axis=0)
    dwm = jnp.where(dw * sdw[None] > 0, jnp.abs(dw), 0.0)
    wt_ref[...] = (rw[None] + dwm).astype(jnp.bfloat16)
    db = b_ref[...] - rb_ref[...]
    sdb = jnp.sum(db, axis=0, keepdims=True)
    dbm_ref[...] = jnp.where(db * sdb > 0, jnp.abs(db), 0.0)

    xm = jnp.mean(x_ref[...], axis=1)
    logits_ref[0] = jax.lax.dot_general(
        xm, wg_ref[...], (((1,), (0,)), ((), ())),
        preferred_element_type=jnp.float32)


_SC_LANES = 16


def _sc_gate_body(logits_hbm, gates_hbm, lg_v):
    c = lax.axis_index("c")
    s = lax.axis_index("s")

    @pl.when(jnp.logical_and(c == 0, s == 0))
    def _():
        pltpu.sync_copy(logits_hbm, lg_v)
        for h in range(_S // _SC_LANES):
            sl = pl.ds(h * _SC_LANES, _SC_LANES)
            lg = [lg_v[e, sl] for e in range(_E)]
            m = lg[0]
            for e in range(1, _E):
                m = jnp.maximum(m, lg[e])
            ex = [jnp.exp(v - m) for v in lg]
            den = ex[0]
            for e in range(1, _E):
                den = den + ex[e]
            for e in range(_E):
                lg_v[e, sl] = ex[e] / den
        pltpu.sync_copy(lg_v, gates_hbm)


def _make_sc_gate(interpret=False):
    return pl.kernel(
        _sc_gate_body,
        out_type=jax.ShapeDtypeStruct((_E, _S), jnp.float32),
        mesh=plsc.VectorSubcoreMesh(core_axis_name="c", subcore_axis_name="s"),
        scratch_types=[
            pltpu.VMEM((_E, _S), jnp.float32),
        ],
        interpret=interpret,
    )


def _moe_body(g_ref, gv_ref, x_ref, wt_ref, dbm_ref, rb_ref,
              out_ref, loss_ref):
    b = pl.program_id(1)

    @pl.when(jnp.logical_and(pl.program_id(0) == 0, b == 0))
    def _():
        gv = gv_ref[...]
        imp = jnp.sum(gv, axis=1, keepdims=True)
        ld = jnp.sum((gv > 0).astype(jnp.float32), axis=1, keepdims=True)

        def cv2(v):
            mean = jnp.sum(v, axis=0, keepdims=True) / _E
            var = jnp.sum((v - mean) ** 2, axis=0, keepdims=True) / (_E - 1)
            return var / (mean * mean + 1e-10)

        loss_ref[...] = (cv2(imp) + cv2(ld)) * 0.001

    g = [[g_ref[e, _N * b + i] for e in range(_E)] for i in range(_NM)]
    gbf = [[v.astype(jnp.bfloat16) for v in row] for row in g]

    def xs(i):
        if i == 0:
            return x_ref[pl.ds(0, 2)].reshape(2 * _T, _D).astype(jnp.bfloat16)
        return x_ref[i + 1].astype(jnp.bfloat16)

    def treesum(terms):
        while len(terms) > 1:
            terms = [terms[j] + terms[j + 1] for j in range(0, len(terms), 2)]
        return terms[0]

    mrows = []
    for i in range(_NM):
        mrows.append(rb_ref[...] + treesum(
            [g[i][e] * dbm_ref[pl.ds(e, 1), :] for e in range(_E)]))
    pieces = []
    for oo in range(_OH // _OO):
        sl = pl.ds(oo * _OO, _OO)
        we = [wt_ref[e, sl, :] for e in range(_E)]
        ys = []
        for i in range(_NM):
            merged = gbf[i][0] * we[0]
            for e in range(1, _E):
                merged = merged + gbf[i][e] * we[e]
            ys.append(jax.lax.dot_general(
                xs(i), merged, (((1,), (1,)), ((), ())),
                preferred_element_type=jnp.float32))
        pieces.append(jnp.concatenate(ys, axis=0))
    y = jnp.concatenate(pieces, axis=1)
    bias_full = jnp.concatenate([mrows[0]] * 2 + mrows[1:], axis=0)
    out_ref[...] = y.reshape(_N, _T, _OH) + bias_full[:, None, :]


def _build_calls(interpret=False):
    prep = pl.pallas_call(
        _prep_body,
        grid=(_KP,),
        in_specs=[
            pl.BlockSpec((_SB, _T, _D), lambda k: (k, 0, 0)),
            pl.BlockSpec((_D, _E), lambda k: (0, 0)),
            pl.BlockSpec((_E, _TP, _D), lambda k: (0, k, 0)),
            pl.BlockSpec((_TP, _D), lambda k: (k, 0)),
            pl.BlockSpec((_E, _TP), lambda k: (0, k)),
            pl.BlockSpec((1, _TP), lambda k: (0, k)),
        ],
        out_specs=[
            pl.BlockSpec((1, _SB, _E), lambda k: (k, 0, 0)),
            pl.BlockSpec((_E, _TP, _D), lambda k: (0, k, 0)),
            pl.BlockSpec((_E, _TP), lambda k: (0, k)),
        ],
        out_shape=[
            jax.ShapeDtypeStruct((_KP, _SB, _E), jnp.float32),
            jax.ShapeDtypeStruct((_E, _O, _D), jnp.bfloat16),
            jax.ShapeDtypeStruct((_E, _O), jnp.float32),
        ],
        interpret=interpret,
    )
    moe = pl.pallas_call(
        _moe_body,
        grid=(_O // _OH, _B),
        in_specs=[
            pl.BlockSpec(memory_space=pltpu.SMEM),
            pl.BlockSpec((_E, _S), lambda oh, b: (0, 0)),
            pl.BlockSpec((_N, _T, _D), lambda oh, b: (b, 0, 0)),
            pl.BlockSpec((_E, _OH, _D), lambda oh, b: (0, oh, 0)),
            pl.BlockSpec((_E, _OH), lambda oh, b: (0, oh)),
            pl.BlockSpec((1, _OH), lambda oh, b: (0, oh)),
        ],
        out_specs=[
            pl.BlockSpec((_N, _T, _OH), lambda oh, b: (b, 0, oh)),
            pl.BlockSpec((1, 1), lambda oh, b: (0, 0)),
        ],
        out_shape=[
            jax.ShapeDtypeStruct((_S, _T, _O), jnp.float32),
            jax.ShapeDtypeStruct((1, 1), jnp.float32),
        ],
        interpret=interpret,
    )
    return prep, moe


_PREP, _MOE = _build_calls()
_SC_GATE = _make_sc_gate()


def kernel(x, w_gate, weight, bias, res_weight, res_bias):
    xc = x.reshape(_S, _T, _D)
    logits, wt, dbm = _PREP(xc, w_gate, weight, res_weight, bias, res_bias)
    gates_t = _SC_GATE(logits.reshape(_S, _E).T)
    out, loss = _MOE(gates_t, gates_t, xc, wt, dbm, res_bias)
    return out.reshape(_B, _L, _O), loss[0, 0]


# speedup vs baseline: 1.0197x; 1.0197x over previous
import functools

import jax
import jax.numpy as jnp
from jax import lax
from jax.experimental import pallas as pl
from jax.experimental.pallas import tpu as pltpu
from jax.experimental.pallas import tpu_sc as plsc

_B, _L, _D, _O, _E, _T = 4, 2048, 1024, 1024, 8, 256
_N = _L // _T
_S = _B * _N
_SB = 8
_TP = 256
_KP = _S // _SB
_OO = 256
_OH = _O // 2
_NM = _N - 1


def _logits_body(x_ref, wg_ref, logits_ref):
    xm = jnp.mean(x_ref[...], axis=1)
    logits_ref[0] = jax.lax.dot_general(
        xm, wg_ref[...], (((1,), (0,)), ((), ())),
        preferred_element_type=jnp.float32)


def _ties_body(w_ref, rw_ref, b_ref, rb_ref, wt_ref, dbm_ref):
    w = w_ref[...]
    rw = rw_ref[...]
    dw = w - rw[None]
    sdw = jnp.sum(dw, axis=0)
    dwm = jnp.where(dw * sdw[None] > 0, jnp.abs(dw), 0.0)
    wt_ref[...] = (rw[None] + dwm).astype(jnp.bfloat16)
    db = b_ref[...] - rb_ref[...]
    sdb = jnp.sum(db, axis=0, keepdims=True)
    dbm_ref[...] = jnp.where(db * sdb > 0, jnp.abs(db), 0.0)


_SC_LANES = 16


def _sc_gate_body(logits_hbm, gates_hbm, lg_v):
    c = lax.axis_index("c")
    s = lax.axis_index("s")

    @pl.when(jnp.logical_and(c == 0, s == 0))
    def _():
        pltpu.sync_copy(logits_hbm, lg_v)
        for h in range(_S // _SC_LANES):
            sl = pl.ds(h * _SC_LANES, _SC_LANES)
            lg = [lg_v[e, sl] for e in range(_E)]
            m = lg[0]
            for e in range(1, _E):
                m = jnp.maximum(m, lg[e])
            ex = [jnp.exp(v - m) for v in lg]
            den = ex[0]
            for e in range(1, _E):
                den = den + ex[e]
            for e in range(_E):
                lg_v[e, sl] = ex[e] / den
        pltpu.sync_copy(lg_v, gates_hbm)


def _make_sc_gate(interpret=False):
    return pl.kernel(
        _sc_gate_body,
        out_type=jax.ShapeDtypeStruct((_E, _S), jnp.float32),
        mesh=plsc.VectorSubcoreMesh(core_axis_name="c", subcore_axis_name="s"),
        scratch_types=[
            pltpu.VMEM((_E, _S), jnp.float32),
        ],
        interpret=interpret,
    )


def _moe_body(g_ref, gv_ref, x_ref, wt_ref, dbm_ref, rb_ref,
              out_ref, loss_ref):
    b = pl.program_id(1)

    @pl.when(jnp.logical_and(pl.program_id(0) == 0, b == 0))
    def _():
        gv = gv_ref[...]
        imp = jnp.sum(gv, axis=1, keepdims=True)
        ld = jnp.sum((gv > 0).astype(jnp.float32), axis=1, keepdims=True)

        def cv2(v):
            mean = jnp.sum(v, axis=0, keepdims=True) / _E
            var = jnp.sum((v - mean) ** 2, axis=0, keepdims=True) / (_E - 1)
            return var / (mean * mean + 1e-10)

        loss_ref[...] = (cv2(imp) + cv2(ld)) * 0.001

    g = [[g_ref[e, _N * b + i] for e in range(_E)] for i in range(_NM)]
    gbf = [[v.astype(jnp.bfloat16) for v in row] for row in g]

    def xs(i):
        if i == 0:
            return x_ref[pl.ds(0, 2)].reshape(2 * _T, _D).astype(jnp.bfloat16)
        return x_ref[i + 1].astype(jnp.bfloat16)

    def treesum(terms):
        while len(terms) > 1:
            terms = [terms[j] + terms[j + 1] for j in range(0, len(terms), 2)]
        return terms[0]

    mrows = []
    for i in range(_NM):
        mrows.append(rb_ref[...] + treesum(
            [g[i][e] * dbm_ref[pl.ds(e, 1), :] for e in range(_E)]))
    pieces = []
    for oo in range(_OH // _OO):
        sl = pl.ds(oo * _OO, _OO)
        we = [wt_ref[e, sl, :] for e in range(_E)]
        ys = []
        for i in range(_NM):
            merged = gbf[i][0] * we[0]
            for e in range(1, _E):
                merged = merged + gbf[i][e] * we[e]
            ys.append(jax.lax.dot_general(
                xs(i), merged, (((1,), (1,)), ((), ())),
                preferred_element_type=jnp.float32))
        pieces.append(jnp.concatenate(ys, axis=0))
    y = jnp.concatenate(pieces, axis=1)
    bias_full = jnp.concatenate([mrows[0]] * 2 + mrows[1:], axis=0)
    out_ref[...] = y.reshape(_N, _T, _OH) + bias_full[:, None, :]


def _build_calls(interpret=False):
    logits_call = pl.pallas_call(
        _logits_body,
        grid=(_KP,),
        in_specs=[
            pl.BlockSpec((_SB, _T, _D), lambda k: (k, 0, 0)),
            pl.BlockSpec((_D, _E), lambda k: (0, 0)),
        ],
        out_specs=pl.BlockSpec((1, _SB, _E), lambda k: (k, 0, 0)),
        out_shape=jax.ShapeDtypeStruct((_KP, _SB, _E), jnp.float32),
        interpret=interpret,
    )
    ties = pl.pallas_call(
        _ties_body,
        grid=(_O // _TP,),
        in_specs=[
            pl.BlockSpec((_E, _TP, _D), lambda k: (0, k, 0)),
            pl.BlockSpec((_TP, _D), lambda k: (k, 0)),
            pl.BlockSpec((_E, _TP), lambda k: (0, k)),
            pl.BlockSpec((1, _TP), lambda k: (0, k)),
        ],
        out_specs=[
            pl.BlockSpec((_E, _TP, _D), lambda k: (0, k, 0)),
            pl.BlockSpec((_E, _TP), lambda k: (0, k)),
        ],
        out_shape=[
            jax.ShapeDtypeStruct((_E, _O, _D), jnp.bfloat16),
            jax.ShapeDtypeStruct((_E, _O), jnp.float32),
        ],
        interpret=interpret,
    )
    moe = pl.pallas_call(
        _moe_body,
        grid=(_O // _OH, _B),
        in_specs=[
            pl.BlockSpec(memory_space=pltpu.SMEM),
            pl.BlockSpec((_E, _S), lambda oh, b: (0, 0)),
            pl.BlockSpec((_N, _T, _D), lambda oh, b: (b, 0, 0)),
            pl.BlockSpec((_E, _OH, _D), lambda oh, b: (0, oh, 0)),
            pl.BlockSpec((_E, _OH), lambda oh, b: (0, oh)),
            pl.BlockSpec((1, _OH), lambda oh, b: (0, oh)),
        ],
        out_specs=[
            pl.BlockSpec((_N, _T, _OH), lambda oh, b: (b, 0, oh)),
            pl.BlockSpec((1, 1), lambda oh, b: (0, 0)),
        ],
        out_shape=[
            jax.ShapeDtypeStruct((_S, _T, _O), jnp.float32),
            jax.ShapeDtypeStruct((1, 1), jnp.float32),
        ],
        interpret=interpret,
    )
    return logits_call, ties, moe


_LOGITS, _TIES, _MOE = _build_calls()
_SC_GATE = _make_sc_gate()


def kernel(x, w_gate, weight, bias, res_weight, res_bias):
    xc = x.reshape(_S, _T, _D)
    logits = _LOGITS(xc, w_gate)
    gates_t = _SC_GATE(logits.reshape(_S, _E).T)
    wt, dbm = _TIES(weight, res_weight, bias, res_bias)
    out, loss = _MOE(gates_t, gates_t, xc, wt, dbm, res_bias)
    return out.reshape(_B, _L, _O), loss[0, 0]


# speedup vs baseline: 1.0210x; 1.0013x over previous
import jax
import jax.numpy as jnp
from jax import lax
from jax.experimental import pallas as pl
from jax.experimental.pallas import tpu as pltpu
from jax.experimental.pallas import tpu_sc as plsc

_B, _L, _D, _O, _E, _T = 4, 2048, 1024, 1024, 8, 256
_N = _L // _T
_S = _B * _N
_SB = 8
_TP = 256
_KP = _S // _SB
_OO = 256
_OH = _O // 2
_NM = _N - 1


def _logits_body(x_ref, wg_ref, logits_ref):
    xm = jnp.mean(x_ref[...], axis=1)
    logits_ref[0] = jax.lax.dot_general(
        xm, wg_ref[...], (((1,), (0,)), ((), ())),
        preferred_element_type=jnp.float32)


def _ties_body(w_ref, rw_ref, b_ref, rb_ref, wt_ref, dbm_ref):
    w = w_ref[...]
    rw = rw_ref[...]
    dw = w - rw[None]
    sdw = jnp.sum(dw, axis=0)
    dwm = jnp.where(dw * sdw[None] > 0, jnp.abs(dw), 0.0)
    wt_ref[...] = (rw[None] + dwm).astype(jnp.bfloat16)
    db = b_ref[...] - rb_ref[...]
    sdb = jnp.sum(db, axis=0, keepdims=True)
    dbm_ref[...] = jnp.where(db * sdb > 0, jnp.abs(db), 0.0)


_SC_LANES = 16


def _sc_gate_body(logits_hbm, gates_hbm, lg_v):
    c = lax.axis_index("c")
    s = lax.axis_index("s")

    @pl.when(jnp.logical_and(c == 0, s == 0))
    def _():
        pltpu.sync_copy(logits_hbm, lg_v)
        for h in range(_S // _SC_LANES):
            sl = pl.ds(h * _SC_LANES, _SC_LANES)
            lg = [lg_v[e, sl] for e in range(_E)]
            m = lg[0]
            for e in range(1, _E):
                m = jnp.maximum(m, lg[e])
            ex = [jnp.exp(v - m) for v in lg]
            den = ex[0]
            for e in range(1, _E):
                den = den + ex[e]
            for e in range(_E):
                lg_v[e, sl] = ex[e] / den
        pltpu.sync_copy(lg_v, gates_hbm)


def _make_sc_gate(interpret=False):
    return pl.kernel(
        _sc_gate_body,
        out_type=jax.ShapeDtypeStruct((_E, _S), jnp.float32),
        mesh=plsc.VectorSubcoreMesh(core_axis_name="c", subcore_axis_name="s"),
        scratch_types=[
            pltpu.VMEM((_E, _S), jnp.float32),
        ],
        interpret=interpret,
    )


def _moe_body(g_ref, gv_ref, x_ref, wt_ref, dbm_ref, rb_ref,
              out_ref, loss_ref):
    b = pl.program_id(1)

    @pl.when(jnp.logical_and(pl.program_id(0) == 0, b == 0))
    def _():
        gv = gv_ref[...]
        imp = jnp.sum(gv, axis=1, keepdims=True)
        ld = jnp.sum((gv > 0).astype(jnp.float32), axis=1, keepdims=True)

        def cv2(v):
            mean = jnp.sum(v, axis=0, keepdims=True) / _E
            var = jnp.sum((v - mean) ** 2, axis=0, keepdims=True) / (_E - 1)
            return var / (mean * mean + 1e-10)

        loss_ref[...] = (cv2(imp) + cv2(ld)) * 0.001

    g = [[g_ref[e, _N * b + i] for e in range(_E)] for i in range(_NM)]
    gbf = [[v.astype(jnp.bfloat16) for v in row] for row in g]

    def xs(i):
        if i == 0:
            return x_ref[pl.ds(0, 2)].reshape(2 * _T, _D).astype(jnp.bfloat16)
        return x_ref[i + 1].astype(jnp.bfloat16)

    def treesum(terms):
        while len(terms) > 1:
            terms = [terms[j] + terms[j + 1] for j in range(0, len(terms), 2)]
        return terms[0]

    mrows = []
    for i in range(_NM):
        mrows.append(rb_ref[...] + treesum(
            [g[i][e] * dbm_ref[pl.ds(e, 1), :] for e in range(_E)]))
    pieces = []
    for oo in range(_OH // _OO):
        sl = pl.ds(oo * _OO, _OO)
        we = [wt_ref[e, sl, :] for e in range(_E)]
        ys = []
        for i in range(_NM):
            merged = gbf[i][0] * we[0]
            for e in range(1, _E):
                merged = merged + gbf[i][e] * we[e]
            ys.append(jax.lax.dot_general(
                xs(i), merged, (((1,), (1,)), ((), ())),
                preferred_element_type=jnp.float32))
        pieces.append(jnp.concatenate(ys, axis=0))
    y = jnp.concatenate(pieces, axis=1)
    bias_full = jnp.concatenate([mrows[0]] * 2 + mrows[1:], axis=0)
    out_ref[...] = y.reshape(_N, _T, _OH) + bias_full[:, None, :]


def _build_calls(interpret=False):
    logits_call = pl.pallas_call(
        _logits_body,
        grid=(_KP,),
        in_specs=[
            pl.BlockSpec((_SB, _T, _D), lambda k: (k, 0, 0)),
            pl.BlockSpec((_D, _E), lambda k: (0, 0)),
        ],
        out_specs=pl.BlockSpec((1, _SB, _E), lambda k: (k, 0, 0)),
        out_shape=jax.ShapeDtypeStruct((_KP, _SB, _E), jnp.float32),
        interpret=interpret,
    )
    ties = pl.pallas_call(
        _ties_body,
        grid=(_O // _TP,),
        in_specs=[
            pl.BlockSpec((_E, _TP, _D), lambda k: (0, k, 0)),
            pl.BlockSpec((_TP, _D), lambda k: (k, 0)),
            pl.BlockSpec((_E, _TP), lambda k: (0, k)),
            pl.BlockSpec((1, _TP), lambda k: (0, k)),
        ],
        out_specs=[
            pl.BlockSpec((_E, _TP, _D), lambda k: (0, k, 0)),
            pl.BlockSpec((_E, _TP), lambda k: (0, k)),
        ],
        out_shape=[
            jax.ShapeDtypeStruct((_E, _O, _D), jnp.bfloat16),
            jax.ShapeDtypeStruct((_E, _O), jnp.float32),
        ],
        interpret=interpret,
    )
    moe = pl.pallas_call(
        _moe_body,
        grid=(_O // _OH, _B),
        in_specs=[
            pl.BlockSpec(memory_space=pltpu.SMEM),
            pl.BlockSpec((_E, _S), lambda oh, b: (0, 0)),
            pl.BlockSpec((_N, _T, _D), lambda oh, b: (b, 0, 0)),
            pl.BlockSpec((_E, _OH, _D), lambda oh, b: (0, oh, 0)),
            pl.BlockSpec((_E, _OH), lambda oh, b: (0, oh)),
            pl.BlockSpec((1, _OH), lambda oh, b: (0, oh)),
        ],
        out_specs=[
            pl.BlockSpec((_N, _T, _OH), lambda oh, b: (b, 0, oh)),
            pl.BlockSpec((1, 1), lambda oh, b: (0, 0)),
        ],
        out_shape=[
            jax.ShapeDtypeStruct((_S, _T, _O), jnp.float32),
            jax.ShapeDtypeStruct((1, 1), jnp.float32),
        ],
        interpret=interpret,
    )
    return logits_call, ties, moe


_LOGITS, _TIES, _MOE = _build_calls()
_SC_GATE = _make_sc_gate()


def kernel(x, w_gate, weight, bias, res_weight, res_bias):
    xc = x.reshape(_S, _T, _D)
    logits = _LOGITS(xc, w_gate)
    gates_t = _SC_GATE(logits.reshape(_S, _E).T)
    wt, dbm = _TIES(weight, res_weight, bias, res_bias)
    out, loss = _MOE(gates_t, gates_t, xc, wt, dbm, res_bias)
    return out.reshape(_B, _L, _O), loss[0, 0]
